# baseline (speedup 1.0000x reference)
import functools

import jax
import jax.numpy as jnp
from jax import lax
from jax.experimental import pallas as pl
from jax.experimental.pallas import tpu as pltpu
from jax.experimental.pallas import tpu_sc as plsc

_D = 64
_CH = 128
_BS = 2
_NW = 32


@functools.lru_cache(maxsize=None)
def _build(n_b, n_t):
    b_per_w = n_b // _NW
    per_w = b_per_w * n_t
    n_super = b_per_w // _BS
    rows_super = _BS * n_t
    chunks = []
    for b in range(_BS):
        o = 0
        while o < n_t:
            l = min(_CH, n_t - o)
            chunks.append((b * n_t + o, b, o, l))
            o += l
    mesh = plsc.VectorSubcoreMesh(core_axis_name="c", subcore_axis_name="s")

    @functools.partial(
        pl.kernel,
        out_type=jax.ShapeDtypeStruct((n_b, n_t, _D), jnp.float32),
        mesh=mesh,
        scratch_types=[
            pltpu.VMEM((per_w,), jnp.int32),
            pltpu.VMEM((_BS, n_t, _D), jnp.float32),
            pltpu.VMEM((_BS, n_t, _D), jnp.float32),
            pltpu.SemaphoreType.DMA,
            pltpu.SemaphoreType.DMA,
        ],
        compiler_params=pltpu.CompilerParams(use_tc_tiling_on_sc=False),
    )
    def gather_kernel(idx_hbm, table_hbm, out_hbm, idx_v, buf0, buf1, g0, g1):
        wid = lax.axis_index("s") * 2 + lax.axis_index("c")
        base = wid * per_w
        b_base = wid * b_per_w
        pltpu.sync_copy(idx_hbm.at[pl.ds(base, per_w)], idx_v)

        def issue(s, buf, sem):
            for (io, b, o, l) in chunks:
                pltpu.async_copy(
                    table_hbm.at[idx_v.at[pl.ds(s * rows_super + io, l)]],
                    buf.at[b, pl.ds(o, l)], sem)

        def drain(s, buf, sem):
            for (io, b, o, l) in chunks:
                pltpu.make_async_copy(
                    table_hbm.at[idx_v.at[pl.ds(s * rows_super + io, l)]],
                    buf.at[b, pl.ds(o, l)], sem).wait()
            pltpu.sync_copy(buf, out_hbm.at[pl.ds(b_base + s * _BS, _BS)])

        issue(0, buf0, g0)

        @pl.loop(0, n_super - 2, step=2)
        def _(s):
            issue(s + 1, buf1, g1)
            drain(s, buf0, g0)
            issue(s + 2, buf0, g0)
            drain(s + 1, buf1, g1)

        issue(n_super - 1, buf1, g1)
        drain(n_super - 2, buf0, g0)
        drain(n_super - 1, buf1, g1)

    return gather_kernel


def kernel(positions, encoding_matrix):
    n_b, n_t = positions.shape
    flat = positions.reshape(-1)
    return _build(n_b, n_t)(flat, encoding_matrix)


# speedup vs baseline: 1.3120x; 1.3120x over previous
import functools

import jax
import jax.numpy as jnp
from jax import lax
from jax.experimental import pallas as pl
from jax.experimental.pallas import tpu as pltpu
from jax.experimental.pallas import tpu_sc as plsc

_D = 64
_CH = 128
_BS = 2
_NW = 32


@functools.lru_cache(maxsize=None)
def _build(n_b, n_t):
    b_per_w = n_b // _NW
    per_w = b_per_w * n_t
    n_super = b_per_w // _BS
    rows_super = _BS * n_t
    chunks = []
    for b in range(_BS):
        o = 0
        while o < n_t:
            l = min(_CH, n_t - o)
            chunks.append((b * n_t + o, b, o, l))
            o += l
    mesh = plsc.VectorSubcoreMesh(core_axis_name="c", subcore_axis_name="s")

    @functools.partial(
        pl.kernel,
        out_type=jax.ShapeDtypeStruct((n_b, n_t, 128), jnp.float32),
        mesh=mesh,
        scratch_types=[
            pltpu.VMEM((per_w,), jnp.int32),
            pltpu.VMEM((_BS, n_t, 128), jnp.float32),
            pltpu.VMEM((_BS, n_t, 128), jnp.float32),
            pltpu.SemaphoreType.DMA,
            pltpu.SemaphoreType.DMA,
        ],
        compiler_params=pltpu.CompilerParams(use_tc_tiling_on_sc=False),
    )
    def gather_kernel(idx_hbm, table_hbm, out_hbm, idx_v, buf0, buf1, g0, g1):
        wid = lax.axis_index("s") * 2 + lax.axis_index("c")
        base = wid * per_w
        b_base = wid * b_per_w
        pltpu.sync_copy(idx_hbm.at[pl.ds(base, per_w)], idx_v)

        def issue(s, buf, sem):
            for (io, b, o, l) in chunks:
                pltpu.async_copy(
                    table_hbm.at[idx_v.at[pl.ds(s * rows_super + io, l)]],
                    buf.at[b, pl.ds(o, l)], sem)

        def drain(s, buf, sem):
            for (io, b, o, l) in chunks:
                pltpu.make_async_copy(
                    table_hbm.at[idx_v.at[pl.ds(s * rows_super + io, l)]],
                    buf.at[b, pl.ds(o, l)], sem).wait()
            pltpu.sync_copy(buf, out_hbm.at[pl.ds(b_base + s * _BS, _BS)])

        issue(0, buf0, g0)

        @pl.loop(0, n_super - 2, step=2)
        def _(s):
            issue(s + 1, buf1, g1)
            drain(s, buf0, g0)
            issue(s + 2, buf0, g0)
            drain(s + 1, buf1, g1)

        issue(n_super - 1, buf1, g1)
        drain(n_super - 2, buf0, g0)
        drain(n_super - 1, buf1, g1)

    return gather_kernel


def kernel(positions, encoding_matrix):
    n_b, n_t = positions.shape
    flat = positions.reshape(-1)
    table128 = jnp.pad(encoding_matrix, ((0, 0), (0, 128 - _D)))
    out128 = _build(n_b, n_t)(flat, table128)
    return out128[:, :, :_D]


# speedup vs baseline: 1.7682x; 1.3478x over previous
import functools

import jax
import jax.numpy as jnp
from jax import lax
from jax.experimental import pallas as pl
from jax.experimental.pallas import tpu as pltpu
from jax.experimental.pallas import tpu_sc as plsc

_D = 64
_CH = 128
_BS = 2
_NW = 32


@functools.lru_cache(maxsize=None)
def _build(n_b, n_t):
    b_per_w = n_b // _NW
    per_w = b_per_w * n_t
    n_super = b_per_w // _BS
    rows_super = _BS * n_t
    chunks = []
    for b in range(_BS):
        o = 0
        while o < n_t:
            l = min(_CH, n_t - o)
            chunks.append((b * n_t + o, b, o, l))
            o += l
    mesh = plsc.VectorSubcoreMesh(core_axis_name="c", subcore_axis_name="s")

    @functools.partial(
        pl.kernel,
        out_type=jax.ShapeDtypeStruct((n_b, n_t, 128), jnp.float32),
        mesh=mesh,
        scratch_types=[
            pltpu.VMEM((per_w,), jnp.int32),
            pltpu.VMEM((_BS, n_t, _D), jnp.float32),
            pltpu.VMEM((_BS, n_t, _D), jnp.float32),
            pltpu.SemaphoreType.DMA,
            pltpu.SemaphoreType.DMA,
        ],
        compiler_params=pltpu.CompilerParams(use_tc_tiling_on_sc=False),
    )
    def gather_kernel(idx_hbm, table_hbm, out_hbm, idx_v, buf0, buf1, g0, g1):
        wid = lax.axis_index("s") * 2 + lax.axis_index("c")
        base = wid * per_w
        b_base = wid * b_per_w
        pltpu.sync_copy(idx_hbm.at[pl.ds(base, per_w)], idx_v)

        def issue(s, buf, sem):
            for (io, b, o, l) in chunks:
                pltpu.async_copy(
                    table_hbm.at[idx_v.at[pl.ds(s * rows_super + io, l)]],
                    buf.at[b, pl.ds(o, l)], sem)

        def drain(s, buf, sem):
            for (io, b, o, l) in chunks:
                pltpu.make_async_copy(
                    table_hbm.at[idx_v.at[pl.ds(s * rows_super + io, l)]],
                    buf.at[b, pl.ds(o, l)], sem).wait()
            pltpu.sync_copy(
                buf, out_hbm.at[pl.ds(b_base + s * _BS, _BS), :, pl.ds(0, _D)])

        issue(0, buf0, g0)

        @pl.loop(0, n_super - 2, step=2)
        def _(s):
            issue(s + 1, buf1, g1)
            drain(s, buf0, g0)
            issue(s + 2, buf0, g0)
            drain(s + 1, buf1, g1)

        issue(n_super - 1, buf1, g1)
        drain(n_super - 2, buf0, g0)
        drain(n_super - 1, buf1, g1)

    return gather_kernel


def kernel(positions, encoding_matrix):
    n_b, n_t = positions.shape
    flat = positions.reshape(-1)
    out128 = _build(n_b, n_t)(flat, encoding_matrix)
    return out128[:, :, :_D]
